# baseline (speedup 1.0000x reference)
import functools

import jax
import jax.numpy as jnp
from jax import lax
from jax.experimental import pallas as pl
from jax.experimental.pallas import tpu as pltpu
from jax.experimental.pallas import tpu_sc as plsc

B, C, D = 1024, 1000, 64
DP = 128
NW = 32
RPW = B // NW
BIG = 3.0e38
CB = 200
NCB = C // CB


def _argmax_body(predt_ref, j0_ref, y0_ref, yacc, jacc):
    i = pl.program_id(0)
    blk = predt_ref[...]
    rows = lax.broadcasted_iota(jnp.int32, (CB, B), 0) + i * CB
    ymax = jnp.max(blk, axis=0, keepdims=True)
    cand = jnp.where(blk == ymax, rows, 2**30)
    jmin = jnp.min(cand, axis=0, keepdims=True)

    @pl.when(i == 0)
    def _():
        yacc[...] = ymax
        jacc[...] = jmin

    @pl.when(i > 0)
    def _():
        better = ymax > yacc[...]
        jacc[...] = jnp.where(better, jmin, jacc[...])
        yacc[...] = jnp.where(better, ymax, yacc[...])

    @pl.when(i == NCB - 1)
    def _():
        j0_ref[...] = jacc[...]
        y0_ref[...] = yacc[...]


def _tc_argmax(predt):
    return pl.pallas_call(
        _argmax_body,
        grid=(NCB,),
        in_specs=[pl.BlockSpec((CB, B), lambda i: (i, 0))],
        out_specs=[pl.BlockSpec((1, B), lambda i: (0, 0)),
                   pl.BlockSpec((1, B), lambda i: (0, 0))],
        out_shape=[jax.ShapeDtypeStruct((1, B), jnp.int32),
                   jax.ShapeDtypeStruct((1, B), jnp.float32)],
        scratch_shapes=[pltpu.VMEM((1, B), jnp.float32),
                        pltpu.VMEM((1, B), jnp.int32)],
    )(predt)


def _sc_body(j0_hbm, w_hbm, wj_hbm, idx_v, rows_v, sem):
    wid = lax.axis_index("s") * 2 + lax.axis_index("c")
    base = wid * RPW
    pltpu.sync_copy(j0_hbm.at[0, pl.ds(base, RPW)], idx_v)
    pltpu.async_copy(w_hbm.at[idx_v], rows_v, sem).wait()
    pltpu.sync_copy(rows_v, wj_hbm.at[pl.ds(base, RPW)])


@functools.lru_cache(maxsize=1)
def _sc_gather():
    return pl.kernel(
        _sc_body,
        out_type=jax.ShapeDtypeStruct((B, DP), jnp.float32),
        mesh=plsc.VectorSubcoreMesh(core_axis_name="c", subcore_axis_name="s"),
        compiler_params=pltpu.CompilerParams(needs_layout_passes=False),
        scratch_types=[
            pltpu.VMEM((RPW,), jnp.int32),
            pltpu.VMEM((RPW, DP), jnp.float32),
            pltpu.SemaphoreType.DMA,
        ],
    )


def _ratio_body(predt_ref, wt3_ref, wj_ref, y0_ref, j0_ref, k_ref, out_ref,
                nj_s, qacc):
    i = pl.program_id(0)
    blk = predt_ref[...]
    y0 = y0_ref[...]
    j0 = j0_ref[...]
    wj = wj_ref[...][:, :D]
    wtc = wt3_ref[0]
    ones = jnp.ones((1, D), jnp.float32)

    @pl.when(i == 0)
    def _():
        nj_s[...] = lax.dot_general(ones, wj * wj, (((1,), (1,)), ((), ())),
                                    preferred_element_type=jnp.float32)

    gt = lax.dot_general(wtc, wj, (((0,), (1,)), ((), ())),
                         preferred_element_type=jnp.float32)
    nc = lax.dot_general(wtc * wtc, ones, (((0,), (1,)), ((), ())),
                         preferred_element_type=jnp.float32)
    d2 = jnp.maximum(nc + nj_s[...] - 2.0 * gt, 0.0)
    rows = lax.broadcasted_iota(jnp.int32, (CB, B), 0) + i * CB
    is_j0 = rows == j0
    m = y0 - blk
    q = jnp.where(is_j0, BIG, (m * m) / jnp.where(is_j0, 1.0, d2))
    qc = jnp.min(q, axis=0, keepdims=True)

    @pl.when(i == 0)
    def _():
        qacc[...] = qc

    @pl.when(i > 0)
    def _():
        qacc[...] = jnp.minimum(qacc[...], qc)

    @pl.when(i == NCB - 1)
    def _():
        out_ref[...] = jnp.sqrt(qacc[...]) / k_ref[0, 0]


def _tc_ratios(predt, wt3, wj, y0, j0, k_smem):
    return pl.pallas_call(
        _ratio_body,
        grid=(NCB,),
        in_specs=[
            pl.BlockSpec((CB, B), lambda i: (i, 0)),
            pl.BlockSpec((1, D, CB), lambda i: (i, 0, 0)),
            pl.BlockSpec((B, DP), lambda i: (0, 0)),
            pl.BlockSpec((1, B), lambda i: (0, 0)),
            pl.BlockSpec((1, B), lambda i: (0, 0)),
            pl.BlockSpec(memory_space=pltpu.SMEM),
        ],
        out_specs=pl.BlockSpec((1, B), lambda i: (0, 0)),
        out_shape=jax.ShapeDtypeStruct((1, B), jnp.float32),
        scratch_shapes=[pltpu.VMEM((1, B), jnp.float32),
                        pltpu.VMEM((1, B), jnp.float32)],
    )(predt, wt3, wj, y0, j0, k_smem)


@jax.jit
def kernel(prediction, target, W, K):
    del target
    predt = prediction.T
    j0, y0 = _tc_argmax(predt)
    w_pad = jnp.pad(W, ((0, 0), (0, DP - D)))
    wj = _sc_gather()(j0, w_pad)
    wt3 = W.T.reshape(D, NCB, CB).transpose(1, 0, 2)
    out = _tc_ratios(predt, wt3, wj, y0, j0, K.reshape(1, 1))
    return out[0]


# speedup vs baseline: 1.0272x; 1.0272x over previous
import functools

import jax
import jax.numpy as jnp
from jax import lax
from jax.experimental import pallas as pl
from jax.experimental.pallas import tpu as pltpu
from jax.experimental.pallas import tpu_sc as plsc

B, C, D = 1024, 1000, 64
DP = 128
NW = 32
RPW = B // NW
BIG = 3.0e38
BLK = 256


def _argmax_body(predt_ref, j0_ref, y0_ref):
    predt = predt_ref[...]
    y0 = jnp.max(predt, axis=0, keepdims=True)
    rows = lax.broadcasted_iota(jnp.int32, (C, BLK), 0)
    cand = jnp.where(predt == y0, rows, 2**30)
    j0_ref[...] = jnp.min(cand, axis=0, keepdims=True)
    y0_ref[...] = y0


def _tc_argmax(predt):
    return pl.pallas_call(
        _argmax_body,
        grid=(B // BLK,),
        in_specs=[pl.BlockSpec((C, BLK), lambda i: (0, i))],
        out_specs=[pl.BlockSpec((1, BLK), lambda i: (0, i)),
                   pl.BlockSpec((1, BLK), lambda i: (0, i))],
        out_shape=[jax.ShapeDtypeStruct((1, B), jnp.int32),
                   jax.ShapeDtypeStruct((1, B), jnp.float32)],
    )(predt)


def _sc_body(j0_hbm, w_hbm, wj_hbm, idx_v, rows_v, sem):
    wid = lax.axis_index("s") * 2 + lax.axis_index("c")
    base = wid * RPW
    pltpu.sync_copy(j0_hbm.at[0, pl.ds(base, RPW)], idx_v)
    pltpu.async_copy(w_hbm.at[idx_v], rows_v, sem).wait()
    pltpu.sync_copy(rows_v, wj_hbm.at[pl.ds(base, RPW)])


@functools.lru_cache(maxsize=1)
def _sc_gather():
    return pl.kernel(
        _sc_body,
        out_type=jax.ShapeDtypeStruct((B, DP), jnp.float32),
        mesh=plsc.VectorSubcoreMesh(core_axis_name="c", subcore_axis_name="s"),
        compiler_params=pltpu.CompilerParams(needs_layout_passes=False),
        scratch_types=[
            pltpu.VMEM((RPW,), jnp.int32),
            pltpu.VMEM((RPW, DP), jnp.float32),
            pltpu.SemaphoreType.DMA,
        ],
    )


def _ratio_body(predt_ref, wt_ref, wj_ref, y0_ref, j0_ref, k_ref, out_ref):
    predt = predt_ref[...]
    y0 = y0_ref[...]
    j0 = j0_ref[...]
    rows = lax.broadcasted_iota(jnp.int32, (C, BLK), 0)
    margins = y0 - predt
    wt = wt_ref[...]
    wj = wj_ref[...][:, :D]
    gt = lax.dot_general(wt, wj, (((0,), (1,)), ((), ())),
                         preferred_element_type=jnp.float32)
    ones = jnp.ones((1, D), jnp.float32)
    nc = lax.dot_general(wt * wt, ones, (((0,), (1,)), ((), ())),
                         preferred_element_type=jnp.float32)
    nj = lax.dot_general(ones, wj * wj, (((1,), (1,)), ((), ())),
                         preferred_element_type=jnp.float32)
    d2 = jnp.maximum(nc + nj - 2.0 * gt, 0.0)
    is_j0 = rows == j0
    q = jnp.where(is_j0, BIG,
                  (margins * margins) / jnp.where(is_j0, 1.0, d2))
    qmin = jnp.min(q, axis=0, keepdims=True)
    out_ref[...] = jnp.sqrt(qmin) / k_ref[0, 0]


def _tc_ratios(predt, wt, wj, y0, j0, k_smem):
    return pl.pallas_call(
        _ratio_body,
        grid=(B // BLK,),
        in_specs=[
            pl.BlockSpec((C, BLK), lambda i: (0, i)),
            pl.BlockSpec((D, C), lambda i: (0, 0)),
            pl.BlockSpec((BLK, DP), lambda i: (i, 0)),
            pl.BlockSpec((1, BLK), lambda i: (0, i)),
            pl.BlockSpec((1, BLK), lambda i: (0, i)),
            pl.BlockSpec(memory_space=pltpu.SMEM),
        ],
        out_specs=pl.BlockSpec((1, BLK), lambda i: (0, i)),
        out_shape=jax.ShapeDtypeStruct((1, B), jnp.float32),
    )(predt, wt, wj, y0, j0, k_smem)


@jax.jit
def kernel(prediction, target, W, K):
    del target
    predt = prediction.T
    j0, y0 = _tc_argmax(predt)
    w_pad = jnp.pad(W, ((0, 0), (0, DP - D)))
    wj = _sc_gather()(j0, w_pad)
    out = _tc_ratios(predt, W.T, wj, y0, j0, K.reshape(1, 1))
    return out[0]
